# baseline (speedup 1.0000x reference)
import functools

import jax
import jax.numpy as jnp
from jax.experimental import pallas as pl
from jax.experimental.pallas import tpu as pltpu

GAMMA = 1.0
BETA = 1.0
_POS_INF_BITS = 0x7F800000


def _encode_body(h_ref, w_ref, b_ref, z_ref):
    z_pre = jax.lax.dot_general(
        h_ref[...].astype(jnp.bfloat16), w_ref[...].astype(jnp.bfloat16),
        (((1,), (0,)), ((), ())),
        preferred_element_type=jnp.float32,
    )
    z_pre = z_pre + b_ref[0]
    z_ref[...] = jnp.maximum(z_pre, 0.0) + BETA * (z_pre > GAMMA).astype(jnp.float32)


def _kth_bits(bits, k):
    t = bits.shape[0]
    hi16 = (bits >> 16).astype(jnp.int16)
    k16 = jnp.int32(k)

    def count_ge(x16, mid16):
        m = (x16 >= mid16).astype(jnp.int16)
        w = m.shape[1]
        while w > 128:
            m = m[:, :w // 2] + m[:, w // 2:]
            w //= 2
        return jnp.sum(m.astype(jnp.int32), axis=1, keepdims=True)

    lo1 = jnp.zeros((t, 1), jnp.int32)
    hi1 = jnp.full((t, 1), 0x7F80, jnp.int32)
    cgt = jnp.zeros((t, 1), jnp.int32)

    def body1(_, carry):
        lo, hi, cgt = carry
        mid = lo + ((hi - lo) >> 1)
        cnt = count_ge(hi16, mid.astype(jnp.int16))
        pred = cnt >= k16
        lo = jnp.where(pred, mid, lo)
        hi = jnp.where(pred, hi, mid)
        cgt = jnp.where(pred, cgt, cnt)
        return lo, hi, cgt

    lo1, hi1, cgt = jax.lax.fori_loop(0, 15, body1, (lo1, hi1, cgt))

    t_hi16 = lo1.astype(jnp.int16)
    k2 = k16 - cgt
    lo_bits = (bits & 0xFFFF) - 32768
    cand = jnp.where(hi16 == t_hi16, lo_bits.astype(jnp.int16),
                     jnp.int16(-32768))

    lo2 = jnp.full((t, 1), -32768, jnp.int32)
    hi2 = jnp.full((t, 1), 32768, jnp.int32)

    def body2(_, carry):
        lo, hi = carry
        mid = lo + ((hi - lo) >> 1)
        cnt = count_ge(cand, mid.astype(jnp.int16))
        pred = cnt >= k2
        lo = jnp.where(pred, mid, lo)
        hi = jnp.where(pred, hi, mid)
        return lo, hi

    lo2, hi2 = jax.lax.fori_loop(0, 16, body2, (lo2, hi2))

    return (lo1 << 16) | (lo2 + 32768)


def _thr_body(z_ref, thr_ref, *, segments, colblock):
    cols = []
    for s, e, k in segments:
        bits = jax.lax.bitcast_convert_type(z_ref[:, s:e], jnp.int32)
        thr = _kth_bits(bits, k)
        cols.extend([thr] * ((e - s) // colblock))
    thr_ref[...] = jnp.concatenate(cols, axis=1)


def _decode_body(z_ref, thr_ref, w_ref, b_ref, out_ref, zs_ref, acc_ref, *, nk):
    k = pl.program_id(1)

    @pl.when(k == 0)
    def _():
        acc_ref[...] = jnp.zeros_like(acc_ref)

    za = z_ref[...]
    bits = jax.lax.bitcast_convert_type(za, jnp.int32)
    ta = thr_ref[...]
    colidx = jax.lax.broadcasted_iota(jnp.int32, ta.shape, 1)
    thrk = jnp.sum(jnp.where(colidx == k, ta, 0), axis=1, keepdims=True)
    zs = jnp.where(bits >= thrk, za, 0.0)
    zs_ref[...] = zs

    acc_ref[...] += jax.lax.dot_general(
        zs.astype(jnp.bfloat16), w_ref[...], (((1,), (0,)), ((), ())),
        preferred_element_type=jnp.float32,
    )

    @pl.when(k == nk - 1)
    def _():
        out_ref[...] = acc_ref[...] + b_ref[0]


def _matryoshka_forward(h, w_enc, b_enc, w_dec, b_dec, levels, topk):
    n, d_in = h.shape
    f = w_enc.shape[1]
    d_out = w_dec.shape[1]

    ta = min(256, n)
    ca = 2048
    grid_a = (f // ca, n // ta)
    z = pl.pallas_call(
        _encode_body,
        grid=grid_a,
        in_specs=[
            pl.BlockSpec((ta, d_in), lambda c, t: (t, 0)),
            pl.BlockSpec((d_in, ca), lambda c, t: (0, c)),
            pl.BlockSpec((1, ca), lambda c, t: (0, c)),
        ],
        out_specs=pl.BlockSpec((ta, ca), lambda c, t: (t, c)),
        out_shape=jax.ShapeDtypeStruct((n, f), jnp.float32),
        compiler_params=pltpu.CompilerParams(
            dimension_semantics=("arbitrary", "arbitrary"),
        ),
    )(h, w_enc, b_enc.reshape(1, f))

    tm = min(128, n)
    kb = 1024
    nk = f // kb
    starts = [0] + levels[:-1]
    segments = tuple(zip(starts, levels, topk))
    thr = pl.pallas_call(
        functools.partial(_thr_body, segments=segments, colblock=kb),
        grid=(n // tm,),
        in_specs=[pl.BlockSpec((tm, f), lambda t: (t, 0))],
        out_specs=pl.BlockSpec((tm, nk), lambda t: (t, 0)),
        out_shape=jax.ShapeDtypeStruct((n, nk), jnp.int32),
        compiler_params=pltpu.CompilerParams(
            dimension_semantics=("arbitrary",),
        ),
    )(z)

    tb = min(1024, n)
    recon, zs = pl.pallas_call(
        functools.partial(_decode_body, nk=nk),
        grid=(n // tb, nk),
        in_specs=[
            pl.BlockSpec((tb, kb), lambda t, k: (t, k)),
            pl.BlockSpec((tb, nk), lambda t, k: (t, 0)),
            pl.BlockSpec((kb, d_out), lambda t, k: (k, 0)),
            pl.BlockSpec((1, d_out), lambda t, k: (0, 0)),
        ],
        out_specs=[
            pl.BlockSpec((tb, d_out), lambda t, k: (t, 0)),
            pl.BlockSpec((tb, kb), lambda t, k: (t, k)),
        ],
        out_shape=[
            jax.ShapeDtypeStruct((n, d_out), jnp.float32),
            jax.ShapeDtypeStruct((n, f), jnp.float32),
        ],
        scratch_shapes=[pltpu.VMEM((tb, d_out), jnp.float32)],
        compiler_params=pltpu.CompilerParams(
            dimension_semantics=("arbitrary", "arbitrary"),
            vmem_limit_bytes=63 * 1024 * 1024,
        ),
    )(z, thr, w_dec.astype(jnp.bfloat16), b_dec.reshape(1, d_out))

    return recon, zs


def kernel(h_2, W_enc, b_enc, W_dec0, b_dec0, W_dec1, b_dec1, W_dec2, b_dec2,
           W_dec3, b_dec3, W_dec4, b_dec4):
    levels = [1024, 2048, 4096, 8192, 16384]
    topk = [8, 16, 32, 64, 128]
    return _matryoshka_forward(h_2, W_enc, b_enc, W_dec4, b_dec4, levels, topk)


# speedup vs baseline: 1.2162x; 1.2162x over previous
import functools

import jax
import jax.numpy as jnp
from jax.experimental import pallas as pl
from jax.experimental.pallas import tpu as pltpu

GAMMA = 1.0
BETA = 1.0
_POS_INF_BITS = 0x7F800000


def _encode_body(h_ref, w_ref, b_ref, z_ref):
    z_pre = jax.lax.dot_general(
        h_ref[...].astype(jnp.bfloat16), w_ref[...].astype(jnp.bfloat16),
        (((1,), (0,)), ((), ())),
        preferred_element_type=jnp.float32,
    )
    z_pre = z_pre + b_ref[0]
    z_ref[...] = jnp.maximum(z_pre, 0.0) + BETA * (z_pre > GAMMA).astype(jnp.float32)


def _count_partial(x16, mid16):
    m = (x16 >= mid16).astype(jnp.int16)
    w = m.shape[1]
    while w > 128:
        m = m[:, :w // 2] + m[:, w // 2:]
        w //= 2
    return m


def _thr_body(z_ref, thr_ref, *, segments, colblock):
    t = z_ref.shape[0]
    ns = len(segments)
    ones = jnp.ones((128, 8), jnp.bfloat16)

    def count(x16, mid):
        m = _count_partial(x16, mid.astype(jnp.int16)).astype(jnp.bfloat16)
        red = jax.lax.dot_general(m, ones, (((1,), (0,)), ((), ())),
                                  preferred_element_type=jnp.float32)
        return red[:, :1]

    bits_l, hi16_l, k_l = [], [], []
    for s, e, k in segments:
        b = jax.lax.bitcast_convert_type(z_ref[:, s:e], jnp.int32)
        bits_l.append(b)
        hi16_l.append((b >> 16).astype(jnp.int16))
        k_l.append(jnp.float32(k))

    lo1 = [jnp.zeros((t, 1), jnp.int32)] * ns
    hi1 = [jnp.full((t, 1), 0x7F80, jnp.int32)] * ns
    cgt = [jnp.zeros((t, 1), jnp.float32)] * ns

    def body1(_, carry):
        lo, hi, cg = carry
        out = ([], [], [])
        for i in range(ns):
            mid = lo[i] + ((hi[i] - lo[i]) >> 1)
            cnt = count(hi16_l[i], mid)
            pred = cnt >= k_l[i]
            out[0].append(jnp.where(pred, mid, lo[i]))
            out[1].append(jnp.where(pred, hi[i], mid))
            out[2].append(jnp.where(pred, cg[i], cnt))
        return out

    lo1, hi1, cgt = jax.lax.fori_loop(0, 15, body1, (lo1, hi1, cgt))

    cand_l, k2_l = [], []
    for i in range(ns):
        lo_bits = (bits_l[i] & 0xFFFF) - 32768
        cand_l.append(jnp.where(hi16_l[i] == lo1[i].astype(jnp.int16),
                                lo_bits.astype(jnp.int16), jnp.int16(-32768)))
        k2_l.append(k_l[i] - cgt[i])

    lo2 = [jnp.full((t, 1), -32768, jnp.int32)] * ns
    hi2 = [jnp.full((t, 1), 32768, jnp.int32)] * ns

    def body2(_, carry):
        lo, hi = carry
        out = ([], [])
        for i in range(ns):
            mid = lo[i] + ((hi[i] - lo[i]) >> 1)
            cnt = count(cand_l[i], mid)
            pred = cnt >= k2_l[i]
            out[0].append(jnp.where(pred, mid, lo[i]))
            out[1].append(jnp.where(pred, hi[i], mid))
        return out

    lo2, hi2 = jax.lax.fori_loop(0, 16, body2, (lo2, hi2))

    cols = []
    for i, (s, e, k) in enumerate(segments):
        thr = (lo1[i] << 16) | (lo2[i] + 32768)
        cols.extend([thr] * ((e - s) // colblock))
    thr_ref[...] = jnp.concatenate(cols, axis=1)


def _decode_body(z_ref, thr_ref, w_ref, b_ref, out_ref, zs_ref, acc_ref, *, nk):
    k = pl.program_id(1)

    @pl.when(k == 0)
    def _():
        acc_ref[...] = jnp.zeros_like(acc_ref)

    za = z_ref[...]
    bits = jax.lax.bitcast_convert_type(za, jnp.int32)
    ta = thr_ref[...]
    colidx = jax.lax.broadcasted_iota(jnp.int32, ta.shape, 1)
    thrk = jnp.sum(jnp.where(colidx == k, ta, 0), axis=1, keepdims=True)
    zs = jnp.where(bits >= thrk, za, 0.0)
    zs_ref[...] = zs

    acc_ref[...] += jax.lax.dot_general(
        zs.astype(jnp.bfloat16), w_ref[...], (((1,), (0,)), ((), ())),
        preferred_element_type=jnp.float32,
    )

    @pl.when(k == nk - 1)
    def _():
        out_ref[...] = acc_ref[...] + b_ref[0]


def _matryoshka_forward(h, w_enc, b_enc, w_dec, b_dec, levels, topk):
    n, d_in = h.shape
    f = w_enc.shape[1]
    d_out = w_dec.shape[1]

    ta = min(256, n)
    ca = 2048
    grid_a = (f // ca, n // ta)
    z = pl.pallas_call(
        _encode_body,
        grid=grid_a,
        in_specs=[
            pl.BlockSpec((ta, d_in), lambda c, t: (t, 0)),
            pl.BlockSpec((d_in, ca), lambda c, t: (0, c)),
            pl.BlockSpec((1, ca), lambda c, t: (0, c)),
        ],
        out_specs=pl.BlockSpec((ta, ca), lambda c, t: (t, c)),
        out_shape=jax.ShapeDtypeStruct((n, f), jnp.float32),
        compiler_params=pltpu.CompilerParams(
            dimension_semantics=("arbitrary", "arbitrary"),
        ),
    )(h, w_enc, b_enc.reshape(1, f))

    tm = min(128, n)
    kb = 1024
    nk = f // kb
    starts = [0] + levels[:-1]
    segments = tuple(zip(starts, levels, topk))
    thr = pl.pallas_call(
        functools.partial(_thr_body, segments=segments, colblock=kb),
        grid=(n // tm,),
        in_specs=[pl.BlockSpec((tm, f), lambda t: (t, 0))],
        out_specs=pl.BlockSpec((tm, nk), lambda t: (t, 0)),
        out_shape=jax.ShapeDtypeStruct((n, nk), jnp.int32),
        compiler_params=pltpu.CompilerParams(
            dimension_semantics=("arbitrary",),
        ),
    )(z)

    tb = min(1024, n)
    recon, zs = pl.pallas_call(
        functools.partial(_decode_body, nk=nk),
        grid=(n // tb, nk),
        in_specs=[
            pl.BlockSpec((tb, kb), lambda t, k: (t, k)),
            pl.BlockSpec((tb, nk), lambda t, k: (t, 0)),
            pl.BlockSpec((kb, d_out), lambda t, k: (k, 0)),
            pl.BlockSpec((1, d_out), lambda t, k: (0, 0)),
        ],
        out_specs=[
            pl.BlockSpec((tb, d_out), lambda t, k: (t, 0)),
            pl.BlockSpec((tb, kb), lambda t, k: (t, k)),
        ],
        out_shape=[
            jax.ShapeDtypeStruct((n, d_out), jnp.float32),
            jax.ShapeDtypeStruct((n, f), jnp.float32),
        ],
        scratch_shapes=[pltpu.VMEM((tb, d_out), jnp.float32)],
        compiler_params=pltpu.CompilerParams(
            dimension_semantics=("arbitrary", "arbitrary"),
            vmem_limit_bytes=63 * 1024 * 1024,
        ),
    )(z, thr, w_dec.astype(jnp.bfloat16), b_dec.reshape(1, d_out))

    return recon, zs


def kernel(h_2, W_enc, b_enc, W_dec0, b_dec0, W_dec1, b_dec1, W_dec2, b_dec2,
           W_dec3, b_dec3, W_dec4, b_dec4):
    levels = [1024, 2048, 4096, 8192, 16384]
    topk = [8, 16, 32, 64, 128]
    return _matryoshka_forward(h_2, W_enc, b_enc, W_dec4, b_dec4, levels, topk)


# speedup vs baseline: 1.2484x; 1.0265x over previous
import functools

import jax
import jax.numpy as jnp
from jax.experimental import pallas as pl
from jax.experimental.pallas import tpu as pltpu

GAMMA = 1.0
BETA = 1.0
_POS_INF_BITS = 0x7F800000


def _encode_body(h_ref, w_ref, b_ref, z_ref):
    z_pre = jax.lax.dot_general(
        h_ref[...].astype(jnp.bfloat16), w_ref[...].astype(jnp.bfloat16),
        (((1,), (0,)), ((), ())),
        preferred_element_type=jnp.float32,
    )
    z_pre = z_pre + b_ref[0]
    z_ref[...] = jnp.maximum(z_pre, 0.0) + BETA * (z_pre > GAMMA).astype(jnp.float32)


def _count_partial(x16, mid16):
    m = (x16 >= mid16).astype(jnp.int16)
    w = m.shape[1]
    while w > 128:
        m = m[:, :w // 2] + m[:, w // 2:]
        w //= 2
    return m


def _thr_body(z_ref, thr_ref, *, segments, colblock):
    t = z_ref.shape[0]
    ns = len(segments)
    ones = jnp.ones((128, 8), jnp.bfloat16)

    def count(x16, mid):
        m = _count_partial(x16, mid.astype(jnp.int16)).astype(jnp.bfloat16)
        red = jax.lax.dot_general(m, ones, (((1,), (0,)), ((), ())),
                                  preferred_element_type=jnp.float32)
        return red[:, :1]

    bits_l, hi16_l, k_l = [], [], []
    for s, e, k in segments:
        b = jax.lax.bitcast_convert_type(z_ref[:, s:e], jnp.int32)
        bits_l.append(b)
        hi16_l.append((b >> 16).astype(jnp.int16))
        k_l.append(jnp.float32(k))

    lo1 = [jnp.zeros((t, 1), jnp.int32)] * ns
    hi1 = [jnp.full((t, 1), 0x7F80, jnp.int32)] * ns
    cgt = [jnp.zeros((t, 1), jnp.float32)] * ns

    def body1(_, carry):
        lo, hi, cg = carry
        out = ([], [], [])
        for i in range(ns):
            mid = lo[i] + ((hi[i] - lo[i]) >> 1)
            cnt = count(hi16_l[i], mid)
            pred = cnt >= k_l[i]
            out[0].append(jnp.where(pred, mid, lo[i]))
            out[1].append(jnp.where(pred, hi[i], mid))
            out[2].append(jnp.where(pred, cg[i], cnt))
        return out

    lo1, hi1, cgt = jax.lax.fori_loop(0, 15, body1, (lo1, hi1, cgt))

    cand_l, k2_l = [], []
    for i in range(ns):
        lo_bits = (bits_l[i] & 0xFFFF) - 32768
        cand_l.append(jnp.where(hi16_l[i] == lo1[i].astype(jnp.int16),
                                lo_bits.astype(jnp.int16), jnp.int16(-32768)))
        k2_l.append(k_l[i] - cgt[i])

    lo2 = [jnp.full((t, 1), -32768, jnp.int32)] * ns
    hi2 = [jnp.full((t, 1), 32768, jnp.int32)] * ns

    def body2(_, carry):
        lo, hi = carry
        out = ([], [])
        for i in range(ns):
            mid = lo[i] + ((hi[i] - lo[i]) >> 1)
            cnt = count(cand_l[i], mid)
            pred = cnt >= k2_l[i]
            out[0].append(jnp.where(pred, mid, lo[i]))
            out[1].append(jnp.where(pred, hi[i], mid))
        return out

    lo2, hi2 = jax.lax.fori_loop(0, 16, body2, (lo2, hi2))

    cols = []
    for i, (s, e, k) in enumerate(segments):
        thr = (lo1[i] << 16) | (lo2[i] + 32768)
        cols.extend([thr] * ((e - s) // colblock))
    thr_ref[...] = jnp.concatenate(cols, axis=1)


def _decode_body(z_ref, thr_ref, w_ref, b_ref, out_ref, zs_ref, acc_ref, *, nk):
    k = pl.program_id(1)

    @pl.when(k == 0)
    def _():
        acc_ref[...] = jnp.zeros_like(acc_ref)

    za = z_ref[...]
    bits = jax.lax.bitcast_convert_type(za, jnp.int32)
    ta = thr_ref[...]
    colidx = jax.lax.broadcasted_iota(jnp.int32, ta.shape, 1)
    thrk = jnp.sum(jnp.where(colidx == k, ta, 0), axis=1, keepdims=True)
    zs = jnp.where(bits >= thrk, za, 0.0)
    zs_ref[...] = zs

    acc_ref[...] += jax.lax.dot_general(
        zs.astype(jnp.bfloat16), w_ref[...], (((1,), (0,)), ((), ())),
        preferred_element_type=jnp.float32,
    )

    @pl.when(k == nk - 1)
    def _():
        out_ref[...] = acc_ref[...] + b_ref[0]


def _matryoshka_forward(h, w_enc, b_enc, w_dec, b_dec, levels, topk):
    n, d_in = h.shape
    f = w_enc.shape[1]
    d_out = w_dec.shape[1]

    ta = min(256, n)
    ca = 2048
    grid_a = (f // ca, n // ta)
    z = pl.pallas_call(
        _encode_body,
        grid=grid_a,
        in_specs=[
            pl.BlockSpec((ta, d_in), lambda c, t: (t, 0)),
            pl.BlockSpec((d_in, ca), lambda c, t: (0, c)),
            pl.BlockSpec((1, ca), lambda c, t: (0, c)),
        ],
        out_specs=pl.BlockSpec((ta, ca), lambda c, t: (t, c)),
        out_shape=jax.ShapeDtypeStruct((n, f), jnp.float32),
        compiler_params=pltpu.CompilerParams(
            dimension_semantics=("arbitrary", "arbitrary"),
        ),
    )(h, w_enc, b_enc.reshape(1, f))

    tm = min(256, n)
    kb = 1024
    nk = f // kb
    starts = [0] + levels[:-1]
    segments = tuple(zip(starts, levels, topk))
    thr = pl.pallas_call(
        functools.partial(_thr_body, segments=segments, colblock=kb),
        grid=(n // tm,),
        in_specs=[pl.BlockSpec((tm, f), lambda t: (t, 0))],
        out_specs=pl.BlockSpec((tm, nk), lambda t: (t, 0)),
        out_shape=jax.ShapeDtypeStruct((n, nk), jnp.int32),
        compiler_params=pltpu.CompilerParams(
            dimension_semantics=("arbitrary",),
            vmem_limit_bytes=63 * 1024 * 1024,
        ),
    )(z)

    tb = min(1024, n)
    recon, zs = pl.pallas_call(
        functools.partial(_decode_body, nk=nk),
        grid=(n // tb, nk),
        in_specs=[
            pl.BlockSpec((tb, kb), lambda t, k: (t, k)),
            pl.BlockSpec((tb, nk), lambda t, k: (t, 0)),
            pl.BlockSpec((kb, d_out), lambda t, k: (k, 0)),
            pl.BlockSpec((1, d_out), lambda t, k: (0, 0)),
        ],
        out_specs=[
            pl.BlockSpec((tb, d_out), lambda t, k: (t, 0)),
            pl.BlockSpec((tb, kb), lambda t, k: (t, k)),
        ],
        out_shape=[
            jax.ShapeDtypeStruct((n, d_out), jnp.float32),
            jax.ShapeDtypeStruct((n, f), jnp.float32),
        ],
        scratch_shapes=[pltpu.VMEM((tb, d_out), jnp.float32)],
        compiler_params=pltpu.CompilerParams(
            dimension_semantics=("arbitrary", "arbitrary"),
            vmem_limit_bytes=63 * 1024 * 1024,
        ),
    )(z, thr, w_dec.astype(jnp.bfloat16), b_dec.reshape(1, d_out))

    return recon, zs


def kernel(h_2, W_enc, b_enc, W_dec0, b_dec0, W_dec1, b_dec1, W_dec2, b_dec2,
           W_dec3, b_dec3, W_dec4, b_dec4):
    levels = [1024, 2048, 4096, 8192, 16384]
    topk = [8, 16, 32, 64, 128]
    return _matryoshka_forward(h_2, W_enc, b_enc, W_dec4, b_dec4, levels, topk)
